# Initial kernel scaffold; baseline (speedup 1.0000x reference)
import jax
import jax.numpy as jnp
from jax.experimental import pallas as pl


def kernel(weight, words, p):
    raise NotImplementedError("write your pallas kernel here")


# speedup vs baseline: 1.4559x; 1.4559x over previous
import functools

import jax
import jax.numpy as jnp
from jax import lax
from jax.experimental import pallas as pl
from jax.experimental.pallas import tpu as pltpu
from jax.experimental.pallas import tpu_sc as plsc

_LANES = 16
_CHUNK = 1024
_GSUB = 128


@functools.partial(jax.jit, static_argnums=(3, 4, 5, 6))
def _masked_gather(weight, mask, words2d, V, D, N, NW):
    NP = N // NW
    n_chunks = NP // _CHUNK
    n_sub = _CHUNK // _GSUB
    info = plsc.get_sparse_core_info()
    nc = info.num_cores

    mesh = plsc.VectorSubcoreMesh(core_axis_name="c", subcore_axis_name="s")

    @functools.partial(
        pl.kernel,
        mesh=mesh,
        out_type=jax.ShapeDtypeStruct((N, D), jnp.float32),
        scratch_types=[
            pltpu.VMEM((n_sub, _GSUB), jnp.int32),
            pltpu.VMEM((_CHUNK,), jnp.float32),
            pltpu.VMEM((_CHUNK, D), jnp.float32),
            pltpu.SemaphoreType.DMA,
        ],
        compiler_params=pltpu.CompilerParams(use_tc_tiling_on_sc=False),
    )
    def gather_kernel(weight_hbm, mask_hbm, words_hbm, out_hbm,
                      idx_v, mval_v, rows_v, sem):
        wid = lax.axis_index("s") * nc + lax.axis_index("c")
        base = wid * NP

        def chunk_body(c, carry):
            start = pl.multiple_of(base + c * _CHUNK, _CHUNK)
            row0 = pl.multiple_of(start // _GSUB, n_sub)
            pltpu.sync_copy(words_hbm.at[pl.ds(row0, n_sub)], idx_v)
            copies = []
            for j in range(n_sub):
                copies.append(pltpu.async_copy(
                    weight_hbm.at[idx_v.at[j]],
                    rows_v.at[pl.ds(j * _GSUB, _GSUB)], sem))
                copies.append(pltpu.async_copy(
                    mask_hbm.at[idx_v.at[j]],
                    mval_v.at[pl.ds(j * _GSUB, _GSUB)], sem))
            for cp in copies:
                cp.wait()

            def row_body(k, carry2):
                mvec = mval_v[pl.ds(k * _LANES, _LANES)]
                for i in range(_LANES):
                    r = k * _LANES + i
                    m = mvec[i]
                    lo = rows_v[r, pl.ds(0, _LANES)]
                    hi = rows_v[r, pl.ds(_LANES, _LANES)]
                    rows_v[r, pl.ds(0, _LANES)] = lo * m
                    rows_v[r, pl.ds(_LANES, _LANES)] = hi * m
                return carry2

            lax.fori_loop(0, _CHUNK // _LANES, row_body, 0)
            pltpu.sync_copy(rows_v, out_hbm.at[pl.ds(start, _CHUNK)])
            return carry

        lax.fori_loop(0, n_chunks, chunk_body, 0)

    return gather_kernel(weight, mask, words2d)


def kernel(weight, words, p):
    V, D = weight.shape
    B, H = words.shape
    N = B * H
    mask = jax.random.bernoulli(
        jax.random.key(42), p, (V,)).astype(jnp.float32) / p
    info = plsc.get_sparse_core_info()
    NW = info.num_cores * info.num_subcores
    words2d = words.reshape(N // _GSUB, _GSUB).astype(jnp.int32)
    out = _masked_gather(weight, mask, words2d, V, D, N, NW)
    return out.reshape(B, H, D)


# speedup vs baseline: 1.4591x; 1.0022x over previous
import functools

import jax
import jax.numpy as jnp
from jax import lax
from jax.experimental import pallas as pl
from jax.experimental.pallas import tpu as pltpu
from jax.experimental.pallas import tpu_sc as plsc

_LANES = 16
_CHUNK = 1024
_GSUB = 128


@functools.partial(jax.jit, static_argnums=(3, 4, 5, 6))
def _masked_gather(weight, mask, wordsf, V, D, N, NW):
    NP = N // NW
    n_chunks = NP // _CHUNK
    n_sub = _CHUNK // _GSUB
    info = plsc.get_sparse_core_info()
    nc = info.num_cores

    mesh = plsc.VectorSubcoreMesh(core_axis_name="c", subcore_axis_name="s")

    @functools.partial(
        pl.kernel,
        mesh=mesh,
        out_type=jax.ShapeDtypeStruct((N * D,), jnp.float32),
        scratch_types=[
            pltpu.VMEM((_CHUNK,), jnp.int32),
            pltpu.VMEM((_CHUNK,), jnp.float32),
            pltpu.VMEM((_CHUNK, D), jnp.float32),
            pltpu.VMEM((_CHUNK * D,), jnp.float32),
            pltpu.SemaphoreType.DMA,
        ],
        compiler_params=pltpu.CompilerParams(use_tc_tiling_on_sc=False),
    )
    def gather_kernel(weight_hbm, mask_hbm, words_hbm, out_hbm,
                      idx_v, mval_v, rows_v, obuf, sem):
        wid = lax.axis_index("s") * nc + lax.axis_index("c")
        base = wid * NP
        wtab = weight_hbm

        def chunk_body(c, carry):
            start = pl.multiple_of(base + c * _CHUNK, _CHUNK)
            pltpu.sync_copy(words_hbm.at[pl.ds(start, _CHUNK)], idx_v)
            copies = []
            for j in range(n_sub):
                sub = idx_v.at[pl.ds(j * _GSUB, _GSUB)]
                copies.append(pltpu.async_copy(
                    wtab.at[sub], rows_v.at[pl.ds(j * _GSUB, _GSUB)], sem))
                copies.append(pltpu.async_copy(
                    mask_hbm.at[sub], mval_v.at[pl.ds(j * _GSUB, _GSUB)], sem))
            for cp in copies:
                cp.wait()

            def row_body(k, carry2):
                mvec = mval_v[pl.ds(k * _LANES, _LANES)]
                o = pl.multiple_of(k * _LANES * D, _LANES * D)
                for i in range(_LANES):
                    r = k * _LANES + i
                    m = mvec[i]
                    lo = rows_v[r, pl.ds(0, _LANES)]
                    hi = rows_v[r, pl.ds(_LANES, _LANES)]
                    obuf[pl.ds(o + i * D, _LANES)] = lo * m
                    obuf[pl.ds(o + i * D + _LANES, _LANES)] = hi * m
                return carry2

            lax.fori_loop(0, _CHUNK // _LANES, row_body, 0)
            pltpu.sync_copy(obuf, out_hbm.at[pl.ds(start * D, _CHUNK * D)])
            return carry

        lax.fori_loop(0, n_chunks, chunk_body, 0)

    return gather_kernel(weight, mask, wordsf)


def kernel(weight, words, p):
    V, D = weight.shape
    B, H = words.shape
    N = B * H
    mask = jax.random.bernoulli(
        jax.random.key(42), p, (V,)).astype(jnp.float32) / p
    info = plsc.get_sparse_core_info()
    NW = info.num_cores * info.num_subcores
    wordsf = words.reshape(N).astype(jnp.int32)
    out = _masked_gather(weight, mask, wordsf, V, D, N, NW)
    return out.reshape(B, H, D)
